# Initial kernel scaffold; baseline (speedup 1.0000x reference)
import jax
import jax.numpy as jnp
from jax.experimental import pallas as pl


def kernel(cls, regr, refi, target_cls, target_regr, target_refi):
    raise NotImplementedError("write your pallas kernel here")


# speedup vs baseline: 1.1727x; 1.1727x over previous
import functools

import jax
import jax.numpy as jnp
from jax import lax
from jax.experimental import pallas as pl
from jax.experimental.pallas import tpu as pltpu
from jax.experimental.pallas import tpu_sc as plsc

N = 1_000_000
CH = 4_000
NCHUNK = N // CH
NW = 16
NV = CH // 16
HR = 128
H = HR * 16
DMIN = -18.0
DMAX = 18.0
SCALE = H / (DMAX - DMIN)
POS_NEG_RATIO = 3.0


def _softplus(d):
    a = jnp.abs(d)
    e = jnp.exp(-a)
    s = e / (e + 2.0)
    z = s * s
    log1p = (2.0 * s) * (1.0 + z * (1.0 / 3.0 + z * (0.2 + z * (1.0 / 7.0))))
    return jnp.maximum(d, 0.0) + log1p


def _splat(x):
    return jnp.full((16,), x, dtype=jnp.float32)


def _body(cls_hbm, lab_hbm, out_hbm, cls_v, lab_v, stage_v, merged_v,
          res_v, hcnt_v, hsum_v, hidx_v, sh_part, sh_hcnt, sh_hsum):
    wid = lax.axis_index("s")
    iota = lax.iota(jnp.int32, 16)
    iota2 = iota * 2
    z16 = jnp.zeros((16,), jnp.float32)

    def zero_stage(i, _):
        stage_v[i, :] = z16
        return 0

    lax.fori_loop(0, 16, zero_stage, 0)

    @pl.when(wid == 0)
    def _():
        pltpu.sync_copy(stage_v, sh_part)

    plsc.subcore_barrier()

    nc = jnp.where(wid < NCHUNK % NW, NCHUNK // NW + 1, NCHUNK // NW)

    def chunk_body(j, carry):
        sp, sn, cp = carry
        c = wid + j * NW
        pltpu.sync_copy(cls_hbm.at[pl.ds(c * (2 * CH), 2 * CH)], cls_v)
        pltpu.sync_copy(lab_hbm.at[pl.ds(c * CH, CH)], lab_v)

        def vec_body(i, carry2):
            sp2, sn2, cp2 = carry2
            b = i * 16
            idx = b * 2 + iota2
            l0 = plsc.load_gather(cls_v, [idx])
            l1 = plsc.load_gather(cls_v, [idx + 1])
            y = lab_v[pl.ds(b, 16)]
            yf = y.astype(jnp.float32)
            m = l1 - l0
            ce = _softplus(m * (1.0 - 2.0 * yf))
            pos = ce * yf
            return (sp2 + pos, sn2 + (ce - pos), cp2 + yf)

        return lax.fori_loop(0, NV, vec_body, (sp, sn, cp))

    sp, sn, cp = lax.fori_loop(0, nc, chunk_body, (z16, z16, z16))

    stage_v[0, :] = sp
    stage_v[1, :] = sn
    stage_v[2, :] = cp
    pltpu.sync_copy(stage_v, sh_part.at[iota], add=True)
    plsc.subcore_barrier()
    pltpu.sync_copy(sh_part, merged_v)

    sum_pos = _splat(jnp.sum(merged_v[0, :]))
    sum_neg = _splat(jnp.sum(merged_v[1, :]))
    n_pos = _splat(jnp.sum(merged_v[2, :]))
    n_neg = float(N) - n_pos
    common = jnp.all(POS_NEG_RATIO * n_pos >= n_neg)

    @pl.when(common & (wid == 0))
    def _():
        res_v[...] = sum_pos / n_pos + sum_neg / n_neg
        pltpu.sync_copy(res_v, out_hbm)

    @pl.when(jnp.logical_not(common))
    def _():
        k = jnp.minimum(n_neg, POS_NEG_RATIO * n_pos)

        def zero_hist(i, _):
            hcnt_v[i, :] = z16
            hsum_v[i, :] = z16
            return 0

        lax.fori_loop(0, HR, zero_hist, 0)

        def fill_hidx(i, _):
            hidx_v[pl.ds(i * 16, 16)] = iota + i * 16
            return 0

        lax.fori_loop(0, HR // 16, fill_hidx, 0)

        @pl.when(wid == 0)
        def _():
            pltpu.sync_copy(hcnt_v, sh_hcnt)
            pltpu.sync_copy(hsum_v, sh_hsum)

        plsc.subcore_barrier()
        ones = jnp.ones((16,), jnp.float32)

        def chunk2(j, _):
            c = wid + j * NW
            pltpu.sync_copy(cls_hbm.at[pl.ds(c * (2 * CH), 2 * CH)], cls_v)
            pltpu.sync_copy(lab_hbm.at[pl.ds(c * CH, CH)], lab_v)

            def vec2(i, _2):
                b = i * 16
                idx = b * 2 + iota2
                l0 = plsc.load_gather(cls_v, [idx])
                l1 = plsc.load_gather(cls_v, [idx + 1])
                y = lab_v[pl.ds(b, 16)]
                neg = y == 0
                m = l1 - l0
                ce = _softplus(m)
                t = jnp.clip((m - DMIN) * SCALE, 0.0, H - 1.0)
                bins = t.astype(jnp.int32)
                row = lax.shift_right_arithmetic(bins, 4)
                col = lax.bitwise_and(bins, 15)
                plsc.addupdate_scatter(hcnt_v, [row, col], ones, mask=neg)
                plsc.addupdate_scatter(hsum_v, [row, col], ce, mask=neg)
                return 0

            return lax.fori_loop(0, NV, vec2, 0)

        lax.fori_loop(0, nc, chunk2, 0)
        pltpu.sync_copy(hcnt_v, sh_hcnt.at[hidx_v], add=True)
        pltpu.sync_copy(hsum_v, sh_hsum.at[hidx_v], add=True)
        plsc.subcore_barrier()

        @pl.when(wid == 0)
        def _():
            pltpu.sync_copy(sh_hcnt, hcnt_v)
            pltpu.sync_copy(sh_hsum, hsum_v)

            def walk(r, carry):
                before, acc = carry
                v = HR - 1 - r
                cvec = lax.rev(hcnt_v[v, :], (0,))
                svec = lax.rev(hsum_v[v, :], (0,))
                cum = jnp.cumsum(cvec)
                cum_excl = before + (cum - cvec)
                take = jnp.clip(k - cum_excl, 0.0, cvec)
                avg = svec / jnp.maximum(cvec, 1.0)
                acc = acc + _splat(jnp.sum(take * avg))
                before = before + _splat(jnp.sum(cvec))
                return (before, acc)

            _, topk_sum = lax.fori_loop(0, HR, walk, (z16, z16))
            res_v[...] = sum_pos / n_pos + topk_sum / k
            pltpu.sync_copy(res_v, out_hbm)


@functools.partial(
    pl.kernel,
    out_type=jax.ShapeDtypeStruct((16,), jnp.float32),
    mesh=plsc.VectorSubcoreMesh(core_axis_name="c", subcore_axis_name="s",
                                num_cores=1),
    compiler_params=pltpu.CompilerParams(needs_layout_passes=False),
    scratch_types=[
        pltpu.VMEM((2 * CH,), jnp.float32),
        pltpu.VMEM((CH,), jnp.int32),
        pltpu.VMEM((16, 16), jnp.float32),
        pltpu.VMEM((16, 16), jnp.float32),
        pltpu.VMEM((16,), jnp.float32),
        pltpu.VMEM((HR, 16), jnp.float32),
        pltpu.VMEM((HR, 16), jnp.float32),
        pltpu.VMEM((HR,), jnp.int32),
        pltpu.VMEM_SHARED((16, 16), jnp.float32),
        pltpu.VMEM_SHARED((HR, 16), jnp.float32),
        pltpu.VMEM_SHARED((HR, 16), jnp.float32),
    ],
)
def _rpn_loss_sc(cls_hbm, lab_hbm, out_hbm, *scratch):
    _body(cls_hbm, lab_hbm, out_hbm, *scratch)


def kernel(cls, regr, refi, target_cls, target_regr, target_refi):
    del regr, refi, target_regr, target_refi
    cls_flat = cls.reshape(2 * N)
    labels = target_cls.reshape(N).astype(jnp.int32)
    out = _rpn_loss_sc(cls_flat, labels)
    return out[0]


# speedup vs baseline: 1.3358x; 1.1391x over previous
import functools

import jax
import jax.numpy as jnp
from jax import lax
from jax.experimental import pallas as pl
from jax.experimental.pallas import tpu as pltpu
from jax.experimental.pallas import tpu_sc as plsc

N = 1_000_000
CH = 4_000
NCHUNK = N // CH
NW = 32
NV = CH // 16
U = 5
HR = 128
H = HR * 16
DMIN = -18.0
DMAX = 18.0
BSCALE = H / (DMAX - DMIN)
POS_NEG_RATIO = 3.0

_P0 = 2.2132784e-05
_P1 = 0.9990102089
_P2 = -0.4891557820
_P3 = 0.2833023836
_P4 = -0.1301179303
_P5 = 0.0301022476

_MESH2 = plsc.VectorSubcoreMesh(core_axis_name="c", subcore_axis_name="s",
                                num_cores=2)
_MESH1 = plsc.VectorSubcoreMesh(core_axis_name="c", subcore_axis_name="s",
                                num_cores=1)
_PARAMS = pltpu.CompilerParams(needs_layout_passes=False)


def _log1p_poly(e):
    return _P0 + e * (_P1 + e * (_P2 + e * (_P3 + e * (_P4 + e * _P5))))


def _gid():
    return lax.axis_index("c") * 16 + lax.axis_index("s")


def _nchunks(g):
    return jnp.where(g < NCHUNK % NW, NCHUNK // NW + 1, NCHUNK // NW)


@functools.partial(
    pl.kernel,
    out_type=jax.ShapeDtypeStruct((NW, 3, 16), jnp.float32),
    mesh=_MESH2,
    compiler_params=_PARAMS,
    scratch_types=[
        pltpu.VMEM((2 * CH,), jnp.float32),
        pltpu.VMEM((CH,), jnp.int32),
        pltpu.VMEM((3, 16), jnp.float32),
    ],
)
def _pass1(cls_hbm, lab_hbm, out_hbm, cls_v, lab_v, stage_v):
    g = _gid()
    iota2 = lax.iota(jnp.int32, 16) * 2
    z16 = jnp.zeros((16,), jnp.float32)

    def chunk_body(j, carry):
        c = g + j * NW
        pltpu.sync_copy(cls_hbm.at[pl.ds(c * (2 * CH), 2 * CH)], cls_v)
        pltpu.sync_copy(lab_hbm.at[pl.ds(c * CH, CH)], lab_v)

        def vec_body(i, accs):
            accs = list(accs)
            for u in range(U):
                sp, st, cp = accs[3 * u:3 * u + 3]
                b = (i * U + u) * 16
                idx = b * 2 + iota2
                l0 = plsc.load_gather(cls_v, [idx])
                l1 = plsc.load_gather(cls_v, [idx + 1])
                y = lab_v[pl.ds(b, 16)]
                yf = y.astype(jnp.float32)
                m = l1 - l0
                a = jnp.abs(m)
                e = jnp.exp(-a)
                ce = 0.5 * (a + m) - m * yf + _log1p_poly(e)
                accs[3 * u] = sp + ce * yf
                accs[3 * u + 1] = st + ce
                accs[3 * u + 2] = cp + yf
            return tuple(accs)

        return lax.fori_loop(0, NV // U, vec_body, carry)

    init = (z16,) * (3 * U)
    accs = lax.fori_loop(0, _nchunks(g), chunk_body, init)
    stage_v[0, :] = sum(accs[0::3], z16)
    stage_v[1, :] = sum(accs[1::3], z16)
    stage_v[2, :] = sum(accs[2::3], z16)
    pltpu.sync_copy(stage_v, out_hbm.at[g])


@functools.partial(
    pl.kernel,
    out_type=(jax.ShapeDtypeStruct((HR, NW, 16), jnp.float32),
              jax.ShapeDtypeStruct((HR, NW, 16), jnp.float32)),
    mesh=_MESH2,
    compiler_params=_PARAMS,
    scratch_types=[
        pltpu.VMEM((2 * CH,), jnp.float32),
        pltpu.VMEM((CH,), jnp.int32),
        pltpu.VMEM((HR, 16), jnp.float32),
        pltpu.VMEM((HR, 16), jnp.float32),
    ],
)
def _hist(cls_hbm, lab_hbm, hcnt_hbm, hsum_hbm, cls_v, lab_v, hcnt_v, hsum_v):
    g = _gid()
    iota2 = lax.iota(jnp.int32, 16) * 2
    z16 = jnp.zeros((16,), jnp.float32)
    ones = jnp.ones((16,), jnp.float32)

    def zero_hist(i, _):
        hcnt_v[i, :] = z16
        hsum_v[i, :] = z16
        return 0

    lax.fori_loop(0, HR, zero_hist, 0)

    def chunk_body(j, _):
        c = g + j * NW
        pltpu.sync_copy(cls_hbm.at[pl.ds(c * (2 * CH), 2 * CH)], cls_v)
        pltpu.sync_copy(lab_hbm.at[pl.ds(c * CH, CH)], lab_v)

        def vec_body(i, _2):
            b = i * 16
            idx = b * 2 + iota2
            l0 = plsc.load_gather(cls_v, [idx])
            l1 = plsc.load_gather(cls_v, [idx + 1])
            y = lab_v[pl.ds(b, 16)]
            neg = y == 0
            m = l1 - l0
            a = jnp.abs(m)
            ce = 0.5 * (a + m) + _log1p_poly(jnp.exp(-a))
            t = jnp.clip((m - DMIN) * BSCALE, 0.0, H - 1.0)
            bins = t.astype(jnp.int32)
            row = lax.shift_right_arithmetic(bins, 4)
            col = lax.bitwise_and(bins, 15)
            plsc.addupdate_scatter(hcnt_v, [row, col], ones, mask=neg)
            plsc.addupdate_scatter(hsum_v, [row, col], ce, mask=neg)
            return 0

        lax.fori_loop(0, NV, vec_body, 0)
        return 0

    lax.fori_loop(0, _nchunks(g), chunk_body, 0)

    def write_row(v, _):
        pltpu.sync_copy(hcnt_v.at[v], hcnt_hbm.at[v, g])
        pltpu.sync_copy(hsum_v.at[v], hsum_hbm.at[v, g])
        return 0

    lax.fori_loop(0, HR, write_row, 0)


@functools.partial(
    pl.kernel,
    out_type=jax.ShapeDtypeStruct((16,), jnp.float32),
    mesh=_MESH1,
    compiler_params=_PARAMS,
    scratch_types=[
        pltpu.VMEM((NW, 16), jnp.float32),
        pltpu.VMEM((NW, 16), jnp.float32),
        pltpu.VMEM((3, 16), jnp.float32),
        pltpu.VMEM((16,), jnp.float32),
    ],
)
def _walk(hcnt_hbm, hsum_hbm, par_hbm, out_hbm, cbuf_v, sbuf_v, par_v, res_v):
    wid = lax.axis_index("s")
    z16 = jnp.zeros((16,), jnp.float32)

    @pl.when(wid == 0)
    def _():
        pltpu.sync_copy(par_hbm, par_v)
        k = par_v[0, :]
        sum_pos = par_v[1, :]
        n_pos = par_v[2, :]

        def walk(r, carry):
            before, acc = carry
            v = HR - 1 - r
            pltpu.sync_copy(hcnt_hbm.at[v], cbuf_v)
            pltpu.sync_copy(hsum_hbm.at[v], sbuf_v)
            cvec = z16
            svec = z16
            for w in range(NW):
                cvec = cvec + cbuf_v[w, :]
                svec = svec + sbuf_v[w, :]
            cvec = lax.rev(cvec, (0,))
            svec = lax.rev(svec, (0,))
            cum = jnp.cumsum(cvec)
            cum_excl = before + (cum - cvec)
            take = jnp.clip(k - cum_excl, 0.0, cvec)
            avg = svec / jnp.maximum(cvec, 1.0)
            acc = acc + jnp.full((16,), jnp.sum(take * avg), jnp.float32)
            before = before + jnp.full((16,), jnp.sum(cvec), jnp.float32)
            return (before, acc)

        _, topk_sum = lax.fori_loop(0, HR, walk, (z16, z16))
        res_v[...] = sum_pos / n_pos + topk_sum / k
        pltpu.sync_copy(res_v, out_hbm)


def kernel(cls, regr, refi, target_cls, target_regr, target_refi):
    del regr, refi, target_regr, target_refi
    cls_flat = cls.reshape(2 * N)
    labels = target_cls.reshape(N).astype(jnp.int32)

    p = _pass1(cls_flat, labels)
    sum_pos = jnp.sum(p[:, 0, :])
    sum_tot = jnp.sum(p[:, 1, :])
    n_pos = jnp.sum(p[:, 2, :])
    sum_neg = sum_tot - sum_pos
    n_neg = jnp.float32(N) - n_pos

    def common_fn(_):
        return sum_pos / n_pos + sum_neg / n_neg

    def rare_fn(_):
        k = jnp.minimum(n_neg, POS_NEG_RATIO * n_pos)
        hcnt, hsum = _hist(cls_flat, labels)
        par = jnp.stack([jnp.full((16,), k, jnp.float32),
                         jnp.full((16,), sum_pos, jnp.float32),
                         jnp.full((16,), n_pos, jnp.float32)])
        return _walk(hcnt, hsum, par)[0]

    return lax.cond(n_neg <= POS_NEG_RATIO * n_pos, common_fn, rare_fn, None)


# speedup vs baseline: 1.3448x; 1.0068x over previous
import functools

import jax
import jax.numpy as jnp
from jax import lax
from jax.experimental import pallas as pl
from jax.experimental.pallas import tpu as pltpu
from jax.experimental.pallas import tpu_sc as plsc

N = 1_000_000
CH = 4_000
NCHUNK = N // CH
NW = 32
NV = CH // 16
U = 4
HR = 128
H = HR * 16
DMIN = -18.0
DMAX = 18.0
BSCALE = H / (DMAX - DMIN)
POS_NEG_RATIO = 3.0

_P0 = 2.2132784e-05
_P1 = 0.9990102089
_P2 = -0.4891557820
_P3 = 0.2833023836
_P4 = -0.1301179303
_P5 = 0.0301022476

_MESH2 = plsc.VectorSubcoreMesh(core_axis_name="c", subcore_axis_name="s",
                                num_cores=2)
_MESH1 = plsc.VectorSubcoreMesh(core_axis_name="c", subcore_axis_name="s",
                                num_cores=1)
_PARAMS = pltpu.CompilerParams(needs_layout_passes=False)


def _log1p_poly(e):
    return _P0 + e * (_P1 + e * (_P2 + e * (_P3 + e * (_P4 + e * _P5))))


def _gid():
    return lax.axis_index("c") * 16 + lax.axis_index("s")


def _nchunks(g):
    return jnp.where(g < NCHUNK % NW, NCHUNK // NW + 1, NCHUNK // NW)


PW = N // NW
SZ = 31256
FULLV = 1952
TAILV = 2
PAD = (FULLV + TAILV) * 16


@functools.partial(
    pl.kernel,
    out_type=jax.ShapeDtypeStruct((NW, 3, 16), jnp.float32),
    mesh=_MESH2,
    compiler_params=_PARAMS,
    scratch_types=[
        pltpu.VMEM((2 * PAD,), jnp.float32),
        pltpu.VMEM((PAD,), jnp.int32),
        pltpu.VMEM((3, 16), jnp.float32),
    ],
)
def _pass1(cls_hbm, lab_hbm, out_hbm, cls_v, lab_v, stage_v):
    g = _gid()
    iota = lax.iota(jnp.int32, 16)
    iota2 = iota * 2
    z16 = jnp.zeros((16,), jnp.float32)

    b = pl.multiple_of(lax.bitwise_and(g * PW, -8), 8)
    e = jnp.where(g == NW - 1, N, lax.bitwise_and((g + 1) * PW, -8))
    ln = e - b
    pltpu.sync_copy(cls_hbm.at[pl.ds(2 * b, 2 * SZ)],
                    cls_v.at[pl.ds(0, 2 * SZ)])
    pltpu.sync_copy(lab_hbm.at[pl.ds(b, SZ)], lab_v.at[pl.ds(0, SZ)])

    def step(bv, valid):
        idx = bv * 2 + iota2
        l0 = plsc.load_gather(cls_v, [idx])
        l1 = plsc.load_gather(cls_v, [idx + 1])
        y = lab_v[pl.ds(bv, 16)]
        yf = y.astype(jnp.float32)
        m = l1 - l0
        if valid is not None:
            m = jnp.where(valid, m, 0.0)
            yf = jnp.where(valid, yf, 0.0)
        a = jnp.abs(m)
        ex = jnp.exp(-a)
        ce = 0.5 * (a + m) - m * yf + _log1p_poly(ex)
        if valid is not None:
            ce = jnp.where(valid, ce, 0.0)
        return ce, yf

    def vec_body(i, accs):
        accs = list(accs)
        for u in range(U):
            sp, st, cp = accs[3 * u:3 * u + 3]
            ce, yf = step((i * U + u) * 16, None)
            accs[3 * u] = sp + ce * yf
            accs[3 * u + 1] = st + ce
            accs[3 * u + 2] = cp + yf
        return tuple(accs)

    accs = lax.fori_loop(0, FULLV // U, vec_body, (z16,) * (3 * U))
    sp = sum(accs[0::3], z16)
    st = sum(accs[1::3], z16)
    cp = sum(accs[2::3], z16)
    for t in range(TAILV):
        bv = (FULLV + t) * 16
        ce, yf = step(bv, bv + iota < ln)
        sp = sp + ce * yf
        st = st + ce
        cp = cp + yf

    stage_v[0, :] = sp
    stage_v[1, :] = st
    stage_v[2, :] = cp
    pltpu.sync_copy(stage_v, out_hbm.at[g])


@functools.partial(
    pl.kernel,
    out_type=(jax.ShapeDtypeStruct((HR, NW, 16), jnp.float32),
              jax.ShapeDtypeStruct((HR, NW, 16), jnp.float32)),
    mesh=_MESH2,
    compiler_params=_PARAMS,
    scratch_types=[
        pltpu.VMEM((2 * CH,), jnp.float32),
        pltpu.VMEM((CH,), jnp.int32),
        pltpu.VMEM((HR, 16), jnp.float32),
        pltpu.VMEM((HR, 16), jnp.float32),
    ],
)
def _hist(cls_hbm, lab_hbm, hcnt_hbm, hsum_hbm, cls_v, lab_v, hcnt_v, hsum_v):
    g = _gid()
    iota2 = lax.iota(jnp.int32, 16) * 2
    z16 = jnp.zeros((16,), jnp.float32)
    ones = jnp.ones((16,), jnp.float32)

    def zero_hist(i, _):
        hcnt_v[i, :] = z16
        hsum_v[i, :] = z16
        return 0

    lax.fori_loop(0, HR, zero_hist, 0)

    def chunk_body(j, _):
        c = g + j * NW
        pltpu.sync_copy(cls_hbm.at[pl.ds(c * (2 * CH), 2 * CH)], cls_v)
        pltpu.sync_copy(lab_hbm.at[pl.ds(c * CH, CH)], lab_v)

        def vec_body(i, _2):
            b = i * 16
            idx = b * 2 + iota2
            l0 = plsc.load_gather(cls_v, [idx])
            l1 = plsc.load_gather(cls_v, [idx + 1])
            y = lab_v[pl.ds(b, 16)]
            neg = y == 0
            m = l1 - l0
            a = jnp.abs(m)
            ce = 0.5 * (a + m) + _log1p_poly(jnp.exp(-a))
            t = jnp.clip((m - DMIN) * BSCALE, 0.0, H - 1.0)
            bins = t.astype(jnp.int32)
            row = lax.shift_right_arithmetic(bins, 4)
            col = lax.bitwise_and(bins, 15)
            plsc.addupdate_scatter(hcnt_v, [row, col], ones, mask=neg)
            plsc.addupdate_scatter(hsum_v, [row, col], ce, mask=neg)
            return 0

        lax.fori_loop(0, NV, vec_body, 0)
        return 0

    lax.fori_loop(0, _nchunks(g), chunk_body, 0)

    def write_row(v, _):
        pltpu.sync_copy(hcnt_v.at[v], hcnt_hbm.at[v, g])
        pltpu.sync_copy(hsum_v.at[v], hsum_hbm.at[v, g])
        return 0

    lax.fori_loop(0, HR, write_row, 0)


@functools.partial(
    pl.kernel,
    out_type=jax.ShapeDtypeStruct((16,), jnp.float32),
    mesh=_MESH1,
    compiler_params=_PARAMS,
    scratch_types=[
        pltpu.VMEM((NW, 16), jnp.float32),
        pltpu.VMEM((NW, 16), jnp.float32),
        pltpu.VMEM((3, 16), jnp.float32),
        pltpu.VMEM((16,), jnp.float32),
    ],
)
def _walk(hcnt_hbm, hsum_hbm, par_hbm, out_hbm, cbuf_v, sbuf_v, par_v, res_v):
    wid = lax.axis_index("s")
    z16 = jnp.zeros((16,), jnp.float32)

    @pl.when(wid == 0)
    def _():
        pltpu.sync_copy(par_hbm, par_v)
        k = par_v[0, :]
        sum_pos = par_v[1, :]
        n_pos = par_v[2, :]

        def walk(r, carry):
            before, acc = carry
            v = HR - 1 - r
            pltpu.sync_copy(hcnt_hbm.at[v], cbuf_v)
            pltpu.sync_copy(hsum_hbm.at[v], sbuf_v)
            cvec = z16
            svec = z16
            for w in range(NW):
                cvec = cvec + cbuf_v[w, :]
                svec = svec + sbuf_v[w, :]
            cvec = lax.rev(cvec, (0,))
            svec = lax.rev(svec, (0,))
            cum = jnp.cumsum(cvec)
            cum_excl = before + (cum - cvec)
            take = jnp.clip(k - cum_excl, 0.0, cvec)
            avg = svec / jnp.maximum(cvec, 1.0)
            acc = acc + jnp.full((16,), jnp.sum(take * avg), jnp.float32)
            before = before + jnp.full((16,), jnp.sum(cvec), jnp.float32)
            return (before, acc)

        _, topk_sum = lax.fori_loop(0, HR, walk, (z16, z16))
        res_v[...] = sum_pos / n_pos + topk_sum / k
        pltpu.sync_copy(res_v, out_hbm)


def kernel(cls, regr, refi, target_cls, target_regr, target_refi):
    del regr, refi, target_regr, target_refi
    cls_flat = cls.reshape(2 * N)
    labels = target_cls.reshape(N).astype(jnp.int32)

    p = _pass1(cls_flat, labels)
    sum_pos = jnp.sum(p[:, 0, :])
    sum_tot = jnp.sum(p[:, 1, :])
    n_pos = jnp.sum(p[:, 2, :])
    sum_neg = sum_tot - sum_pos
    n_neg = jnp.float32(N) - n_pos

    def common_fn(_):
        return sum_pos / n_pos + sum_neg / n_neg

    def rare_fn(_):
        k = jnp.minimum(n_neg, POS_NEG_RATIO * n_pos)
        hcnt, hsum = _hist(cls_flat, labels)
        par = jnp.stack([jnp.full((16,), k, jnp.float32),
                         jnp.full((16,), sum_pos, jnp.float32),
                         jnp.full((16,), n_pos, jnp.float32)])
        return _walk(hcnt, hsum, par)[0]

    return lax.cond(n_neg <= POS_NEG_RATIO * n_pos, common_fn, rare_fn, None)


# speedup vs baseline: 14.0607x; 10.4554x over previous
import functools

import jax
import jax.numpy as jnp
from jax import lax
from jax.experimental import pallas as pl
from jax.experimental.pallas import tpu as pltpu
from jax.experimental.pallas import tpu_sc as plsc

N = 1_000_000
CH = 4_000
NCHUNK = N // CH
NW = 32
NV = CH // 16
U = 4
HR = 128
H = HR * 16
DMIN = -18.0
DMAX = 18.0
BSCALE = H / (DMAX - DMIN)
POS_NEG_RATIO = 3.0

_P0 = 2.2132784e-05
_P1 = 0.9990102089
_P2 = -0.4891557820
_P3 = 0.2833023836
_P4 = -0.1301179303
_P5 = 0.0301022476

_MESH2 = plsc.VectorSubcoreMesh(core_axis_name="c", subcore_axis_name="s",
                                num_cores=2)
_MESH1 = plsc.VectorSubcoreMesh(core_axis_name="c", subcore_axis_name="s",
                                num_cores=1)
_PARAMS = pltpu.CompilerParams(needs_layout_passes=False)


def _log1p_poly(e):
    return _P0 + e * (_P1 + e * (_P2 + e * (_P3 + e * (_P4 + e * _P5))))


def _gid():
    return lax.axis_index("c") * 16 + lax.axis_index("s")


def _nchunks(g):
    return jnp.where(g < NCHUNK % NW, NCHUNK // NW + 1, NCHUNK // NW)


PW = N // NW
SZ = 31256
FULLV = 1952
TAILV = 2
PAD = (FULLV + TAILV) * 16


@functools.partial(
    pl.kernel,
    out_type=jax.ShapeDtypeStruct((NW, 3, 16), jnp.float32),
    mesh=_MESH2,
    compiler_params=_PARAMS,
    scratch_types=[
        pltpu.VMEM((PAD,), jnp.float32),
        pltpu.VMEM((PAD,), jnp.float32),
        pltpu.VMEM((PAD,), jnp.int32),
        pltpu.VMEM((3, 16), jnp.float32),
    ],
)
def _pass1(l0_hbm, l1_hbm, lab_hbm, out_hbm, l0_v, l1_v, lab_v, stage_v):
    g = _gid()
    iota = lax.iota(jnp.int32, 16)
    z16 = jnp.zeros((16,), jnp.float32)

    b = pl.multiple_of(lax.bitwise_and(g * PW, -8), 8)
    e = jnp.where(g == NW - 1, N, lax.bitwise_and((g + 1) * PW, -8))
    ln = e - b
    pltpu.sync_copy(l0_hbm.at[pl.ds(b, SZ)], l0_v.at[pl.ds(0, SZ)])
    pltpu.sync_copy(l1_hbm.at[pl.ds(b, SZ)], l1_v.at[pl.ds(0, SZ)])
    pltpu.sync_copy(lab_hbm.at[pl.ds(b, SZ)], lab_v.at[pl.ds(0, SZ)])

    def step(bv, valid):
        l0 = l0_v[pl.ds(bv, 16)]
        l1 = l1_v[pl.ds(bv, 16)]
        y = lab_v[pl.ds(bv, 16)]
        yf = y.astype(jnp.float32)
        m = l1 - l0
        if valid is not None:
            m = jnp.where(valid, m, 0.0)
            yf = jnp.where(valid, yf, 0.0)
        a = jnp.abs(m)
        ex = jnp.exp(-a)
        ce = 0.5 * (a + m) - m * yf + _log1p_poly(ex)
        if valid is not None:
            ce = jnp.where(valid, ce, 0.0)
        return ce, yf

    def vec_body(i, accs):
        accs = list(accs)
        for u in range(U):
            sp, st, cp = accs[3 * u:3 * u + 3]
            ce, yf = step((i * U + u) * 16, None)
            accs[3 * u] = sp + ce * yf
            accs[3 * u + 1] = st + ce
            accs[3 * u + 2] = cp + yf
        return tuple(accs)

    accs = lax.fori_loop(0, FULLV // U, vec_body, (z16,) * (3 * U))
    sp = sum(accs[0::3], z16)
    st = sum(accs[1::3], z16)
    cp = sum(accs[2::3], z16)
    for t in range(TAILV):
        bv = (FULLV + t) * 16
        ce, yf = step(bv, bv + iota < ln)
        sp = sp + ce * yf
        st = st + ce
        cp = cp + yf

    stage_v[0, :] = sp
    stage_v[1, :] = st
    stage_v[2, :] = cp
    pltpu.sync_copy(stage_v, out_hbm.at[g])


@functools.partial(
    pl.kernel,
    out_type=(jax.ShapeDtypeStruct((HR, NW, 16), jnp.float32),
              jax.ShapeDtypeStruct((HR, NW, 16), jnp.float32)),
    mesh=_MESH2,
    compiler_params=_PARAMS,
    scratch_types=[
        pltpu.VMEM((CH,), jnp.float32),
        pltpu.VMEM((CH,), jnp.float32),
        pltpu.VMEM((CH,), jnp.int32),
        pltpu.VMEM((HR, 16), jnp.float32),
        pltpu.VMEM((HR, 16), jnp.float32),
    ],
)
def _hist(l0_hbm, l1_hbm, lab_hbm, hcnt_hbm, hsum_hbm, l0_v, l1_v, lab_v,
          hcnt_v, hsum_v):
    g = _gid()
    z16 = jnp.zeros((16,), jnp.float32)
    ones = jnp.ones((16,), jnp.float32)

    def zero_hist(i, _):
        hcnt_v[i, :] = z16
        hsum_v[i, :] = z16
        return 0

    lax.fori_loop(0, HR, zero_hist, 0)

    def chunk_body(j, _):
        c = g + j * NW
        pltpu.sync_copy(l0_hbm.at[pl.ds(c * CH, CH)], l0_v)
        pltpu.sync_copy(l1_hbm.at[pl.ds(c * CH, CH)], l1_v)
        pltpu.sync_copy(lab_hbm.at[pl.ds(c * CH, CH)], lab_v)

        def vec_body(i, _2):
            b = i * 16
            l0 = l0_v[pl.ds(b, 16)]
            l1 = l1_v[pl.ds(b, 16)]
            y = lab_v[pl.ds(b, 16)]
            neg = y == 0
            m = l1 - l0
            a = jnp.abs(m)
            ce = 0.5 * (a + m) + _log1p_poly(jnp.exp(-a))
            t = jnp.clip((m - DMIN) * BSCALE, 0.0, H - 1.0)
            bins = t.astype(jnp.int32)
            row = lax.shift_right_arithmetic(bins, 4)
            col = lax.bitwise_and(bins, 15)
            plsc.addupdate_scatter(hcnt_v, [row, col], ones, mask=neg)
            plsc.addupdate_scatter(hsum_v, [row, col], ce, mask=neg)
            return 0

        lax.fori_loop(0, NV, vec_body, 0)
        return 0

    lax.fori_loop(0, _nchunks(g), chunk_body, 0)

    def write_row(v, _):
        pltpu.sync_copy(hcnt_v.at[v], hcnt_hbm.at[v, g])
        pltpu.sync_copy(hsum_v.at[v], hsum_hbm.at[v, g])
        return 0

    lax.fori_loop(0, HR, write_row, 0)


@functools.partial(
    pl.kernel,
    out_type=jax.ShapeDtypeStruct((16,), jnp.float32),
    mesh=_MESH1,
    compiler_params=_PARAMS,
    scratch_types=[
        pltpu.VMEM((NW, 16), jnp.float32),
        pltpu.VMEM((NW, 16), jnp.float32),
        pltpu.VMEM((3, 16), jnp.float32),
        pltpu.VMEM((16,), jnp.float32),
    ],
)
def _walk(hcnt_hbm, hsum_hbm, par_hbm, out_hbm, cbuf_v, sbuf_v, par_v, res_v):
    wid = lax.axis_index("s")
    z16 = jnp.zeros((16,), jnp.float32)

    @pl.when(wid == 0)
    def _():
        pltpu.sync_copy(par_hbm, par_v)
        k = par_v[0, :]
        sum_pos = par_v[1, :]
        n_pos = par_v[2, :]

        def walk(r, carry):
            before, acc = carry
            v = HR - 1 - r
            pltpu.sync_copy(hcnt_hbm.at[v], cbuf_v)
            pltpu.sync_copy(hsum_hbm.at[v], sbuf_v)
            cvec = z16
            svec = z16
            for w in range(NW):
                cvec = cvec + cbuf_v[w, :]
                svec = svec + sbuf_v[w, :]
            cvec = lax.rev(cvec, (0,))
            svec = lax.rev(svec, (0,))
            cum = jnp.cumsum(cvec)
            cum_excl = before + (cum - cvec)
            take = jnp.clip(k - cum_excl, 0.0, cvec)
            avg = svec / jnp.maximum(cvec, 1.0)
            acc = acc + jnp.full((16,), jnp.sum(take * avg), jnp.float32)
            before = before + jnp.full((16,), jnp.sum(cvec), jnp.float32)
            return (before, acc)

        _, topk_sum = lax.fori_loop(0, HR, walk, (z16, z16))
        res_v[...] = sum_pos / n_pos + topk_sum / k
        pltpu.sync_copy(res_v, out_hbm)


def kernel(cls, regr, refi, target_cls, target_regr, target_refi):
    del regr, refi, target_regr, target_refi
    l0 = cls[0, :, 0]
    l1 = cls[0, :, 1]
    labels = target_cls.reshape(N).astype(jnp.int32)

    p = _pass1(l0, l1, labels)
    sum_pos = jnp.sum(p[:, 0, :])
    sum_tot = jnp.sum(p[:, 1, :])
    n_pos = jnp.sum(p[:, 2, :])
    sum_neg = sum_tot - sum_pos
    n_neg = jnp.float32(N) - n_pos

    def common_fn(_):
        return sum_pos / n_pos + sum_neg / n_neg

    def rare_fn(_):
        k = jnp.minimum(n_neg, POS_NEG_RATIO * n_pos)
        hcnt, hsum = _hist(l0, l1, labels)
        par = jnp.stack([jnp.full((16,), k, jnp.float32),
                         jnp.full((16,), sum_pos, jnp.float32),
                         jnp.full((16,), n_pos, jnp.float32)])
        return _walk(hcnt, hsum, par)[0]

    return lax.cond(n_neg <= POS_NEG_RATIO * n_pos, common_fn, rare_fn, None)
